# baseline (speedup 1.0000x reference)
import functools

import jax
import jax.numpy as jnp
from jax import lax
from jax.experimental import pallas as pl
from jax.experimental.pallas import tpu as pltpu
from jax.experimental.pallas import tpu_sc as plsc

VOCAB = 120
EMBED_DIM = 200
CHUNK = 128
NBUF = 4


def _sc_gather(idx3, table, B):
    info = plsc.get_sparse_core_info()
    NC = info.num_cores
    n_chunks = idx3.shape[1]
    b_per_w = n_chunks * CHUNK
    mesh = plsc.VectorSubcoreMesh(core_axis_name="c", subcore_axis_name="s")

    @functools.partial(
        pl.kernel,
        mesh=mesh,
        compiler_params=pltpu.CompilerParams(use_tc_tiling_on_sc=False),
        out_type=jax.ShapeDtypeStruct((B, EMBED_DIM), jnp.float32),
        scratch_types=[
            pltpu.VMEM((n_chunks, CHUNK), jnp.int32),
            [pltpu.VMEM((CHUNK, EMBED_DIM), jnp.float32)] * NBUF,
            [pltpu.SemaphoreType.DMA] * NBUF,
            [pltpu.SemaphoreType.DMA] * NBUF,
        ],
    )
    def k(idx_hbm, table_hbm, out_hbm, idx_v, rows, gsem, wsem):
        wid = lax.axis_index("s") * NC + lax.axis_index("c")
        base = wid * b_per_w
        pltpu.sync_copy(idx_hbm.at[wid], idx_v)

        pltpu.async_copy(table_hbm.at[idx_v.at[0]], rows[0], gsem[0])
        pltpu.async_copy(table_hbm.at[idx_v.at[1]], rows[1], gsem[1])

        def body(i, carry):
            for s4 in range(NBUF):
                j = NBUF * i + s4
                cur = rows[s4]
                nxt = rows[(s4 + 2) % NBUF]

                @pl.when(j + 2 < n_chunks)
                def _():
                    @pl.when(j >= 2)
                    def _():
                        pltpu.make_async_copy(
                            nxt, out_hbm.at[pl.ds(0, CHUNK)], wsem[(s4 + 2) % NBUF]
                        ).wait()

                    pltpu.async_copy(
                        table_hbm.at[idx_v.at[j + 2]], nxt, gsem[(s4 + 2) % NBUF]
                    )

                pltpu.make_async_copy(
                    table_hbm.at[idx_v.at[j]], cur, gsem[s4]
                ).wait()
                pltpu.async_copy(
                    cur, out_hbm.at[pl.ds(base + j * CHUNK, CHUNK)], wsem[s4]
                )
            return carry

        lax.fori_loop(0, n_chunks // NBUF, body, 0, unroll=False)
        for s4 in range(NBUF):
            pltpu.make_async_copy(
                rows[s4], out_hbm.at[pl.ds(0, CHUNK)], wsem[s4]
            ).wait()

    return k(idx3, table)


def kernel(x, atom_fea):
    orig_shape = x.shape
    B = x.size
    info = plsc.get_sparse_core_info()
    NW = info.num_cores * info.num_subcores
    n_chunks = B // (NW * CHUNK)
    idx3 = x.astype(jnp.int32).reshape(NW, n_chunks, CHUNK)
    out = _sc_gather(idx3, atom_fea, B)
    return out.reshape(*orig_shape, EMBED_DIM)


# speedup vs baseline: 1.0019x; 1.0019x over previous
import functools

import jax
import jax.numpy as jnp
from jax import lax
from jax.experimental import pallas as pl
from jax.experimental.pallas import tpu as pltpu
from jax.experimental.pallas import tpu_sc as plsc

VOCAB = 120
EMBED_DIM = 200
NBUF = 4


def _sc_gather(idx3, table):
    info = plsc.get_sparse_core_info()
    NC = info.num_cores
    NW = NC * info.num_subcores
    rows_per_w, chunk = idx3.shape[1], idx3.shape[2]
    n_rows = NW * rows_per_w
    mesh = plsc.VectorSubcoreMesh(core_axis_name="c", subcore_axis_name="s")

    @functools.partial(
        pl.kernel,
        mesh=mesh,
        compiler_params=pltpu.CompilerParams(use_tc_tiling_on_sc=False),
        out_type=jax.ShapeDtypeStruct((n_rows, chunk, EMBED_DIM), jnp.float32),
        scratch_types=[
            pltpu.VMEM((rows_per_w, chunk), jnp.int32),
            [pltpu.VMEM((chunk, EMBED_DIM), jnp.float32)] * NBUF,
            [pltpu.SemaphoreType.DMA] * NBUF,
            [pltpu.SemaphoreType.DMA] * NBUF,
        ],
    )
    def k(idx_hbm, table_hbm, out_hbm, idx_v, rows, gsem, wsem):
        wid = lax.axis_index("s") * NC + lax.axis_index("c")
        base = wid * rows_per_w
        pltpu.sync_copy(idx_hbm.at[wid], idx_v)

        pltpu.async_copy(table_hbm.at[idx_v.at[0]], rows[0], gsem[0])
        pltpu.async_copy(table_hbm.at[idx_v.at[1]], rows[1], gsem[1])

        def body(i, carry):
            for s4 in range(NBUF):
                j = NBUF * i + s4
                cur = rows[s4]
                nxt = rows[(s4 + 2) % NBUF]

                @pl.when(j + 2 < rows_per_w)
                def _():
                    @pl.when(j >= 2)
                    def _():
                        pltpu.make_async_copy(
                            nxt, out_hbm.at[0], wsem[(s4 + 2) % NBUF]
                        ).wait()

                    pltpu.async_copy(
                        table_hbm.at[idx_v.at[j + 2]], nxt, gsem[(s4 + 2) % NBUF]
                    )

                pltpu.make_async_copy(
                    table_hbm.at[idx_v.at[j]], cur, gsem[s4]
                ).wait()
                pltpu.async_copy(cur, out_hbm.at[base + j], wsem[s4])
            return carry

        lax.fori_loop(0, rows_per_w // NBUF, body, 0, unroll=False)
        for s4 in range(NBUF):
            pltpu.make_async_copy(rows[s4], out_hbm.at[0], wsem[s4]).wait()

    return k(idx3, table)


def kernel(x, atom_fea):
    n_rows, chunk = x.shape
    info = plsc.get_sparse_core_info()
    NW = info.num_cores * info.num_subcores
    idx3 = x.astype(jnp.int32).reshape(NW, n_rows // NW, chunk)
    return _sc_gather(idx3, atom_fea)


# speedup vs baseline: 1.3675x; 1.3649x over previous
import functools

import jax
import jax.numpy as jnp
from jax import lax
from jax.experimental import pallas as pl
from jax.experimental.pallas import tpu as pltpu
from jax.experimental.pallas import tpu_sc as plsc

VOCAB = 120
EMBED_DIM = 200
LANE = 128
TAIL = EMBED_DIM - LANE
NBUF = 4
TC_ROWS = 128


def _sc_head(idx3, table_a, n_rows, chunk):
    info = plsc.get_sparse_core_info()
    NC = info.num_cores
    rows_per_w = idx3.shape[1]
    mesh = plsc.VectorSubcoreMesh(core_axis_name="c", subcore_axis_name="s")

    @functools.partial(
        pl.kernel,
        mesh=mesh,
        out_type=jax.ShapeDtypeStruct((n_rows, chunk, EMBED_DIM), jnp.float32),
        scratch_types=[
            pltpu.VMEM((rows_per_w, chunk), jnp.int32),
            [pltpu.VMEM((chunk, LANE), jnp.float32)] * NBUF,
            [pltpu.SemaphoreType.DMA] * NBUF,
            [pltpu.SemaphoreType.DMA] * NBUF,
        ],
    )
    def k(idx_hbm, table_hbm, out_hbm, idx_v, rows, gsem, wsem):
        wid = lax.axis_index("s") * NC + lax.axis_index("c")
        base = wid * rows_per_w
        pltpu.sync_copy(idx_hbm.at[wid], idx_v)

        pltpu.async_copy(table_hbm.at[idx_v.at[0]], rows[0], gsem[0])
        pltpu.async_copy(table_hbm.at[idx_v.at[1]], rows[1], gsem[1])

        def body(i, carry):
            for s in range(NBUF):
                j = NBUF * i + s
                cur = rows[s]
                nxt = rows[(s + 2) % NBUF]

                @pl.when(j + 2 < rows_per_w)
                def _():
                    @pl.when(j >= 2)
                    def _():
                        pltpu.make_async_copy(
                            nxt,
                            out_hbm.at[0].at[:, pl.ds(0, LANE)],
                            wsem[(s + 2) % NBUF],
                        ).wait()

                    pltpu.async_copy(
                        table_hbm.at[idx_v.at[j + 2]], nxt, gsem[(s + 2) % NBUF]
                    )

                pltpu.make_async_copy(
                    table_hbm.at[idx_v.at[j]], cur, gsem[s]
                ).wait()
                pltpu.async_copy(
                    cur, out_hbm.at[base + j].at[:, pl.ds(0, LANE)], wsem[s]
                )
            return carry

        lax.fori_loop(0, rows_per_w // NBUF, body, 0, unroll=False)
        for s in range(NBUF):
            pltpu.make_async_copy(
                rows[s], out_hbm.at[0].at[:, pl.ds(0, LANE)], wsem[s]
            ).wait()

    return k(idx3, table_a)


def _tc_tail(x_flat, table_b, out1, n_rows, chunk):
    grid = n_rows // TC_ROWS

    def body(x_ref, tb_ref, _, out_ref, vals_ref, sem):
        i = pl.program_id(0)
        idx = x_ref[...]
        onehot = (
            idx[:, :, None]
            == lax.broadcasted_iota(jnp.int32, (TC_ROWS, chunk, LANE), 2)
        ).astype(jnp.float32)
        vals_ref[...] = lax.dot_general(
            onehot,
            tb_ref[...],
            dimension_numbers=(((2,), (0,)), ((), ())),
            preferred_element_type=jnp.float32,
        )
        copy = pltpu.make_async_copy(
            vals_ref.at[:, :, pl.ds(LANE, TAIL)],
            out_ref.at[pl.ds(i * TC_ROWS, TC_ROWS), :, pl.ds(LANE, TAIL)],
            sem,
        )
        copy.start()
        copy.wait()

    return pl.pallas_call(
        body,
        grid=(grid,),
        in_specs=[
            pl.BlockSpec((TC_ROWS, chunk), lambda i: (i, 0)),
            pl.BlockSpec((LANE, EMBED_DIM), lambda i: (0, 0)),
            pl.BlockSpec(memory_space=pltpu.HBM),
        ],
        out_specs=pl.BlockSpec(memory_space=pltpu.HBM),
        out_shape=jax.ShapeDtypeStruct((n_rows, chunk, EMBED_DIM), jnp.float32),
        scratch_shapes=[
            pltpu.VMEM((TC_ROWS, chunk, EMBED_DIM), jnp.float32),
            pltpu.SemaphoreType.DMA,
        ],
        input_output_aliases={2: 0},
    )(x_flat, table_b, out1)


def kernel(x, atom_fea):
    n_rows, chunk = x.shape
    info = plsc.get_sparse_core_info()
    NW = info.num_cores * info.num_subcores
    xi = x.astype(jnp.int32)
    idx3 = xi.reshape(NW, n_rows // NW, chunk)
    table_a = atom_fea[:, :LANE]
    table_b = jnp.zeros((LANE, EMBED_DIM), jnp.float32)
    table_b = lax.dynamic_update_slice(
        table_b, atom_fea[:, LANE:EMBED_DIM], (0, LANE)
    )
    out1 = _sc_head(idx3, table_a, n_rows, chunk)
    return _tc_tail(xi, table_b, out1, n_rows, chunk)


# speedup vs baseline: 1.5317x; 1.1201x over previous
import functools

import jax
import jax.numpy as jnp
from jax import lax
from jax.experimental import pallas as pl
from jax.experimental.pallas import tpu as pltpu
from jax.experimental.pallas import tpu_sc as plsc

VOCAB = 120
EMBED_DIM = 200
LANE = 128
TAIL = EMBED_DIM - LANE
NBUF = 4
TC_ROWS = 128


def _sc_head_inplace(out_ref, idx3, table_a):
    info = plsc.get_sparse_core_info()
    NC = info.num_cores
    rows_per_w = idx3.shape[1]
    mesh = plsc.VectorSubcoreMesh(core_axis_name="c", subcore_axis_name="s")

    @functools.partial(
        pl.kernel,
        mesh=mesh,
        out_type=(),
        scratch_types=[
            pltpu.VMEM((rows_per_w, idx3.shape[2]), jnp.int32),
            [pltpu.VMEM((idx3.shape[2], LANE), jnp.float32)] * NBUF,
            [pltpu.SemaphoreType.DMA] * NBUF,
            [pltpu.SemaphoreType.DMA] * NBUF,
        ],
    )
    def k(idx_hbm, table_hbm, out_hbm, idx_v, rows, gsem, wsem):
        wid = lax.axis_index("s") * NC + lax.axis_index("c")
        base = wid * rows_per_w
        pltpu.sync_copy(idx_hbm.at[wid], idx_v)

        pltpu.async_copy(table_hbm.at[idx_v.at[0]], rows[0], gsem[0])
        pltpu.async_copy(table_hbm.at[idx_v.at[1]], rows[1], gsem[1])

        def body(i, carry):
            for s in range(NBUF):
                j = NBUF * i + s
                cur = rows[s]
                nxt = rows[(s + 2) % NBUF]

                @pl.when(j + 2 < rows_per_w)
                def _():
                    @pl.when(j >= 2)
                    def _():
                        pltpu.make_async_copy(
                            nxt,
                            out_hbm.at[0].at[:, pl.ds(0, LANE)],
                            wsem[(s + 2) % NBUF],
                        ).wait()

                    pltpu.async_copy(
                        table_hbm.at[idx_v.at[j + 2]], nxt, gsem[(s + 2) % NBUF]
                    )

                pltpu.make_async_copy(
                    table_hbm.at[idx_v.at[j]], cur, gsem[s]
                ).wait()
                pltpu.async_copy(
                    cur, out_hbm.at[base + j].at[:, pl.ds(0, LANE)], wsem[s]
                )
            return carry

        lax.fori_loop(0, rows_per_w // NBUF, body, 0, unroll=False)
        for s in range(NBUF):
            pltpu.make_async_copy(
                rows[s], out_hbm.at[0].at[:, pl.ds(0, LANE)], wsem[s]
            ).wait()

    k(idx3, table_a, out_ref)


def _tc_tail(x, table_b, n_rows, chunk):
    grid = n_rows // TC_ROWS

    def body(x_ref, tb_ref, out_ref, vals_ref, sem):
        i = pl.program_id(0)
        idx = x_ref[...]
        onehot = (
            idx[:, :, None]
            == lax.broadcasted_iota(jnp.int32, (TC_ROWS, chunk, LANE), 2)
        ).astype(jnp.float32)
        vals_ref[...] = lax.dot_general(
            onehot,
            tb_ref[...],
            dimension_numbers=(((2,), (0,)), ((), ())),
            preferred_element_type=jnp.float32,
        )
        copy = pltpu.make_async_copy(
            vals_ref.at[:, :, pl.ds(LANE, TAIL)],
            out_ref.at[pl.ds(i * TC_ROWS, TC_ROWS), :, pl.ds(LANE, TAIL)],
            sem,
        )
        copy.start()
        copy.wait()

    return pl.pallas_call(
        body,
        grid=(grid,),
        in_specs=[
            pl.BlockSpec((TC_ROWS, chunk), lambda i: (i, 0)),
            pl.BlockSpec((LANE, EMBED_DIM), lambda i: (0, 0)),
        ],
        out_specs=pl.BlockSpec(memory_space=pltpu.HBM),
        out_shape=jax.ShapeDtypeStruct((n_rows, chunk, EMBED_DIM), jnp.float32),
        scratch_shapes=[
            pltpu.VMEM((TC_ROWS, chunk, EMBED_DIM), jnp.float32),
            pltpu.SemaphoreType.DMA,
        ],
    )(x, table_b)


def kernel(x, atom_fea):
    n_rows, chunk = x.shape
    info = plsc.get_sparse_core_info()
    NW = info.num_cores * info.num_subcores
    xi = x.astype(jnp.int32)
    idx3 = xi.reshape(NW, n_rows // NW, chunk)
    table_a = atom_fea[:, :LANE]
    table_b = jnp.zeros((LANE, EMBED_DIM), jnp.float32)
    table_b = lax.dynamic_update_slice(
        table_b, atom_fea[:, LANE:EMBED_DIM], (0, LANE)
    )
    out = _tc_tail(xi, table_b, n_rows, chunk)
    out_ref = jax.new_ref(out)
    _sc_head_inplace(out_ref, idx3, table_a)
    return out_ref[...]


# speedup vs baseline: 1.5321x; 1.0002x over previous
import functools

import jax
import jax.numpy as jnp
from jax import lax
from jax.experimental import pallas as pl
from jax.experimental.pallas import tpu as pltpu
from jax.experimental.pallas import tpu_sc as plsc

VOCAB = 120
EMBED_DIM = 200
LANE = 128
TAIL = EMBED_DIM - LANE
NBUF = 4
TC_ROWS = 128


def _sc_head_inplace(out_ref, idx3, table_a):
    info = plsc.get_sparse_core_info()
    NC = info.num_cores
    rows_per_w = idx3.shape[1]
    mesh = plsc.VectorSubcoreMesh(core_axis_name="c", subcore_axis_name="s")

    @functools.partial(
        pl.kernel,
        mesh=mesh,
        out_type=(),
        scratch_types=[
            pltpu.VMEM((rows_per_w, idx3.shape[2]), jnp.int32),
            [pltpu.VMEM((idx3.shape[2], LANE), jnp.float32)] * NBUF,
            [pltpu.SemaphoreType.DMA] * NBUF,
            [pltpu.SemaphoreType.DMA] * NBUF,
        ],
    )
    def k(idx_hbm, table_hbm, out_hbm, idx_v, rows, gsem, wsem):
        wid = lax.axis_index("s") * NC + lax.axis_index("c")
        base = wid * rows_per_w
        pltpu.sync_copy(idx_hbm.at[wid], idx_v)

        pltpu.async_copy(table_hbm.at[idx_v.at[0]], rows[0], gsem[0])
        pltpu.async_copy(table_hbm.at[idx_v.at[1]], rows[1], gsem[1])

        def body(i, carry):
            for s in range(NBUF):
                j = NBUF * i + s
                cur = rows[s]
                nxt = rows[(s + 2) % NBUF]

                @pl.when(j + 2 < rows_per_w)
                def _():
                    @pl.when(j >= 2)
                    def _():
                        pltpu.make_async_copy(
                            nxt,
                            out_hbm.at[0].at[:, pl.ds(0, LANE)],
                            wsem[(s + 2) % NBUF],
                        ).wait()

                    pltpu.async_copy(
                        table_hbm.at[idx_v.at[j + 2]], nxt, gsem[(s + 2) % NBUF]
                    )

                pltpu.make_async_copy(
                    table_hbm.at[idx_v.at[j]], cur, gsem[s]
                ).wait()
                pltpu.async_copy(
                    cur, out_hbm.at[base + j].at[:, pl.ds(0, LANE)], wsem[s]
                )
            return carry

        lax.fori_loop(0, rows_per_w // NBUF, body, 0, unroll=False)
        for s in range(NBUF):
            pltpu.make_async_copy(
                rows[s], out_hbm.at[0].at[:, pl.ds(0, LANE)], wsem[s]
            ).wait()

    k(idx3, table_a, out_ref)


def _tc_tail(x, table_b, n_rows, chunk):
    grid = n_rows // TC_ROWS

    def body(x_ref, tb_ref, out_ref, vals_ref, sem):
        i = pl.program_id(0)
        idx = x_ref[...]
        onehot = (
            idx[:, :, None]
            == lax.broadcasted_iota(jnp.int32, (TC_ROWS, chunk, LANE), 2)
        ).astype(jnp.float32)
        vals_ref[...] = lax.dot_general(
            onehot,
            tb_ref[...],
            dimension_numbers=(((2,), (0,)), ((), ())),
            preferred_element_type=jnp.float32,
        )
        copy = pltpu.make_async_copy(
            vals_ref.at[:, :, pl.ds(LANE, TAIL)],
            out_ref.at[pl.ds(i * TC_ROWS, TC_ROWS), :, pl.ds(LANE, TAIL)],
            sem,
        )
        copy.start()
        copy.wait()

    return pl.pallas_call(
        body,
        grid=(grid,),
        in_specs=[
            pl.BlockSpec((TC_ROWS, chunk), lambda i: (i, 0)),
            pl.BlockSpec((LANE, EMBED_DIM), lambda i: (0, 0)),
        ],
        out_specs=pl.BlockSpec(memory_space=pltpu.HBM),
        out_shape=jax.ShapeDtypeStruct((n_rows, chunk, EMBED_DIM), jnp.float32),
        scratch_shapes=[
            pltpu.VMEM((TC_ROWS, chunk, EMBED_DIM), jnp.float32),
            pltpu.SemaphoreType.DMA,
        ],
    )(x, table_b)


def kernel(x, atom_fea):
    n_rows, chunk = x.shape
    info = plsc.get_sparse_core_info()
    NW = info.num_cores * info.num_subcores
    xi = x.astype(jnp.int32)
    idx3 = xi.reshape(NW, n_rows // NW, chunk)
    table_a = atom_fea[:, :LANE]
    table_b = jnp.zeros((LANE, EMBED_DIM), jnp.float32)
    table_b = lax.dynamic_update_slice(
        table_b, atom_fea[:, LANE:EMBED_DIM], (0, LANE)
    )
    out = _tc_tail(xi, table_b, n_rows, chunk)
    out_ref = jax.new_ref(out)
    _sc_head_inplace(out_ref, idx3, table_a)
    return jax.freeze(out_ref)


# speedup vs baseline: 2.2703x; 1.4818x over previous
import functools

import jax
import jax.numpy as jnp
from jax import lax
from jax.experimental import pallas as pl
from jax.experimental.pallas import tpu as pltpu
from jax.experimental.pallas import tpu_sc as plsc

VOCAB = 120
EMBED_DIM = 200
LANE = 128
TAIL = EMBED_DIM - LANE
NBUF = 4
TC_ROWS = 128


def _sc_head_inplace(out_ref, idx3, table_a):
    info = plsc.get_sparse_core_info()
    NC = info.num_cores
    rows_per_w = idx3.shape[1]
    mesh = plsc.VectorSubcoreMesh(core_axis_name="c", subcore_axis_name="s")

    @functools.partial(
        pl.kernel,
        mesh=mesh,
        out_type=(),
        scratch_types=[
            pltpu.VMEM((rows_per_w, idx3.shape[2]), jnp.int32),
            pltpu.VMEM_SHARED((VOCAB, LANE), jnp.float32),
            [pltpu.VMEM((idx3.shape[2], LANE), jnp.float32)] * NBUF,
            [pltpu.SemaphoreType.DMA] * NBUF,
            [pltpu.SemaphoreType.DMA] * NBUF,
        ],
    )
    def k(idx_hbm, table_hbm, out_hbm, idx_v, tbl_sh, rows, gsem, wsem):
        sid = lax.axis_index("s")
        wid = sid * NC + lax.axis_index("c")
        base = wid * rows_per_w

        @pl.when(sid == 0)
        def _():
            pltpu.sync_copy(table_hbm, tbl_sh)

        pltpu.sync_copy(idx_hbm.at[wid], idx_v)
        plsc.subcore_barrier()

        pltpu.async_copy(tbl_sh.at[idx_v.at[0]], rows[0], gsem[0])
        pltpu.async_copy(tbl_sh.at[idx_v.at[1]], rows[1], gsem[1])

        def body(i, carry):
            for s in range(NBUF):
                j = NBUF * i + s
                cur = rows[s]
                nxt = rows[(s + 2) % NBUF]

                @pl.when(j + 2 < rows_per_w)
                def _():
                    @pl.when(j >= 2)
                    def _():
                        pltpu.make_async_copy(
                            nxt,
                            out_hbm.at[0].at[:, pl.ds(0, LANE)],
                            wsem[(s + 2) % NBUF],
                        ).wait()

                    pltpu.async_copy(
                        tbl_sh.at[idx_v.at[j + 2]], nxt, gsem[(s + 2) % NBUF]
                    )

                pltpu.make_async_copy(
                    tbl_sh.at[idx_v.at[j]], cur, gsem[s]
                ).wait()
                pltpu.async_copy(
                    cur, out_hbm.at[base + j].at[:, pl.ds(0, LANE)], wsem[s]
                )
            return carry

        lax.fori_loop(0, rows_per_w // NBUF, body, 0, unroll=False)
        for s in range(NBUF):
            pltpu.make_async_copy(
                rows[s], out_hbm.at[0].at[:, pl.ds(0, LANE)], wsem[s]
            ).wait()

    k(idx3, table_a, out_ref)


def _tc_tail(x, table_b, n_rows, chunk):
    grid = n_rows // TC_ROWS

    def body(x_ref, tb_ref, out_ref, vals_ref, sem):
        i = pl.program_id(0)
        idx = x_ref[...]
        onehot = (
            idx[:, :, None]
            == lax.broadcasted_iota(jnp.int32, (TC_ROWS, chunk, LANE), 2)
        ).astype(jnp.float32)
        vals_ref[...] = lax.dot_general(
            onehot,
            tb_ref[...],
            dimension_numbers=(((2,), (0,)), ((), ())),
            preferred_element_type=jnp.float32,
        )
        copy = pltpu.make_async_copy(
            vals_ref.at[:, :, pl.ds(LANE, TAIL)],
            out_ref.at[pl.ds(i * TC_ROWS, TC_ROWS), :, pl.ds(LANE, TAIL)],
            sem,
        )
        copy.start()
        copy.wait()

    return pl.pallas_call(
        body,
        grid=(grid,),
        in_specs=[
            pl.BlockSpec((TC_ROWS, chunk), lambda i: (i, 0)),
            pl.BlockSpec((LANE, EMBED_DIM), lambda i: (0, 0)),
        ],
        out_specs=pl.BlockSpec(memory_space=pltpu.HBM),
        out_shape=jax.ShapeDtypeStruct((n_rows, chunk, EMBED_DIM), jnp.float32),
        scratch_shapes=[
            pltpu.VMEM((TC_ROWS, chunk, EMBED_DIM), jnp.float32),
            pltpu.SemaphoreType.DMA,
        ],
    )(x, table_b)


def kernel(x, atom_fea):
    n_rows, chunk = x.shape
    info = plsc.get_sparse_core_info()
    NW = info.num_cores * info.num_subcores
    xi = x.astype(jnp.int32)
    idx3 = xi.reshape(NW, n_rows // NW, chunk)
    table_a = atom_fea[:, :LANE]
    table_b = jnp.zeros((LANE, EMBED_DIM), jnp.float32)
    table_b = lax.dynamic_update_slice(
        table_b, atom_fea[:, LANE:EMBED_DIM], (0, LANE)
    )
    out = _tc_tail(xi, table_b, n_rows, chunk)
    out_ref = jax.new_ref(out)
    _sc_head_inplace(out_ref, idx3, table_a)
    return jax.freeze(out_ref)
